# baseline (speedup 1.0000x reference)
import functools

import jax
import jax.numpy as jnp
from jax import lax
from jax.experimental import pallas as pl
from jax.experimental.pallas import tpu as pltpu
from jax.experimental.pallas import tpu_sc as plsc

VOCAB = 1000000
EMBED_DIM = 64
BATCH = 16384
HIST_LEN = 50

_B = BATCH * HIST_LEN
_NW = 32
_BPW = _B // _NW
_CHUNK = 100
_GPS = 4
_STEP = _CHUNK * _GPS
_BSTEP = _STEP // HIST_LEN
_NSTEPS = _BPW // _STEP
_NBUF = 2
_ROWS_PER_W = _BPW // _CHUNK

_mesh = plsc.VectorSubcoreMesh(core_axis_name="c", subcore_axis_name="s")


@functools.partial(
    pl.kernel,
    mesh=_mesh,
    out_type=jax.ShapeDtypeStruct((BATCH, HIST_LEN, EMBED_DIM), jnp.float32),
    scratch_types=[
        pltpu.VMEM((_ROWS_PER_W, _CHUNK), jnp.int32),
        [pltpu.VMEM((_STEP, EMBED_DIM), jnp.float32) for _ in range(_NBUF)],
        [pltpu.SemaphoreType.DMA for _ in range(_NBUF)],
        [pltpu.SemaphoreType.DMA for _ in range(_NBUF)],
    ],
    compiler_params=pltpu.CompilerParams(use_tc_tiling_on_sc=False),
)
def _gather_kernel(table_hbm, idx_hbm, out_hbm, idx_v, rows_bufs, gsems, ssems):
    wid = lax.axis_index("s") * 2 + lax.axis_index("c")
    b_base = wid * (BATCH // _NW)
    pltpu.sync_copy(idx_hbm.at[pl.ds(wid * _ROWS_PER_W, _ROWS_PER_W)], idx_v)

    def fire_gathers(step, p):
        for i in range(_GPS):
            pltpu.async_copy(
                table_hbm.at[idx_v.at[step * _GPS + i]],
                rows_bufs[p].at[pl.ds(i * _CHUNK, _CHUNK)],
                gsems[p],
            )

    def wait_gathers(p):
        pltpu.make_async_copy(
            table_hbm.at[pl.ds(0, _STEP)], rows_bufs[p], gsems[p]).wait()

    def wait_stores(p):
        pltpu.make_async_copy(
            table_hbm.at[pl.ds(0, _STEP)], rows_bufs[p], ssems[p]).wait()

    def fire_stores(step, p):
        b0 = b_base + step * _BSTEP
        for b in range(_BSTEP):
            pltpu.async_copy(
                rows_bufs[p].at[pl.ds(b * HIST_LEN, HIST_LEN)],
                out_hbm.at[b0 + b],
                ssems[p],
            )

    def retire(step, p):
        wait_gathers(p)
        fire_stores(step, p)
        wait_stores(p)

    for p in range(_NBUF):
        fire_gathers(p, p)

    def outer(t, carry):
        for p in range(_NBUF):
            step = t * _NBUF + p
            retire(step, p)
            fire_gathers(step + _NBUF, p)
        return carry

    lax.fori_loop(0, _NSTEPS // _NBUF - 1, outer, 0)

    for p in range(_NBUF):
        retire(_NSTEPS - _NBUF + p, p)


def kernel(item_embeddings, batch_data):
    idx = batch_data.reshape(-1).astype(jnp.int32)
    idx2d = idx.reshape(_B // _CHUNK, _CHUNK)
    return _gather_kernel(item_embeddings, idx2d)
